# baseline (speedup 1.0000x reference)
import functools

import jax
import jax.numpy as jnp
from jax.experimental import pallas as pl


def _silu(x):
    return x * jax.nn.sigmoid(x)


_BE = 2000


def _edge_mlp_body(g_ref, a_ref, we1a_ref, we2_ref, be2_ref, wx1_ref,
                   bx1_ref, wx2_ref, bx2_ref, m_ref, w_ref):
    g = g_ref[...]
    a = a_ref[...]
    m1 = _silu(g + a * we1a_ref[...])
    m = _silu(jnp.dot(m1, we2_ref[...],
                      preferred_element_type=jnp.float32) + be2_ref[...])
    t = _silu(jnp.dot(m, wx1_ref[...],
                      preferred_element_type=jnp.float32) + bx1_ref[...])
    w = jnp.dot(t, wx2_ref[...],
                preferred_element_type=jnp.float32) + bx2_ref[...]
    m_ref[...] = m
    w_ref[...] = w


@functools.partial(jax.jit, static_argnames=("interpret",))
def _edge_mlp(g, a, we1a, we2, be2, wx1, bx1, wx2, bx2, interpret=False):
    e = g.shape[0]
    h = g.shape[1]
    grid = (e // _BE,)
    full = lambda i: (0, 0)
    return pl.pallas_call(
        _edge_mlp_body,
        grid=grid,
        in_specs=[
            pl.BlockSpec((_BE, h), lambda i: (i, 0)),
            pl.BlockSpec((_BE, 1), lambda i: (i, 0)),
            pl.BlockSpec((1, h), full),
            pl.BlockSpec((h, h), full),
            pl.BlockSpec((1, h), full),
            pl.BlockSpec((h, h), full),
            pl.BlockSpec((1, h), full),
            pl.BlockSpec((h, 1), full),
            pl.BlockSpec((1, 1), full),
        ],
        out_specs=[
            pl.BlockSpec((_BE, h), lambda i: (i, 0)),
            pl.BlockSpec((_BE, 1), lambda i: (i, 0)),
        ],
        out_shape=[
            jax.ShapeDtypeStruct((e, h), jnp.float32),
            jax.ShapeDtypeStruct((e, 1), jnp.float32),
        ],
        interpret=interpret,
    )(g, a.reshape(e, 1), we1a.reshape(1, h), we2, be2.reshape(1, h),
      wx1, bx1.reshape(1, h), wx2, bx2.reshape(1, 1))


def kernel(atom_nums, coords, masses, masses_normalized, cond_labels,
           cond_mask, moments, temb, edge_index, embed, W_h, b_h, gamma,
           beta, We1, be1, We2, be2, Wx1, bx1, Wx2, bx2, Wh1, bh1, Wh2, bh2,
           interpret=False):
    n = coords.shape[0]
    src = edge_index[0]
    dst = edge_index[1]
    aemb = jnp.take(embed, atom_nums, axis=0)
    f = jnp.concatenate([aemb, temb, masses / 12.0, masses_normalized,
                         cond_labels, cond_mask, moments], axis=-1)
    h = f @ W_h + b_h
    a = jnp.sum((coords[src] - coords[dst]) ** 2, axis=-1)
    x = coords
    num_layers = gamma.shape[0]
    hdim = h.shape[1]
    c = 1.0 / float(edge_index.shape[1] // n)
    for i in range(num_layers):
        mu = h.mean(axis=-1, keepdims=True)
        var = h.var(axis=-1, keepdims=True)
        hn = (h - mu) / jnp.sqrt(var + 1e-5) * gamma[i] + beta[i]
        p = hn @ We1[i][:hdim]
        q = hn @ We1[i][hdim:2 * hdim] + be1[i]
        g = p[src] + q[dst]
        m, w = _edge_mlp(g, a, We1[i][2 * hdim], We2[i], be2[i],
                         Wx1[i], bx1[i], Wx2[i], bx2[i], interpret=interpret)
        delta = (x[dst] - x[src]) * w
        x = x + c * jax.ops.segment_sum(delta, dst, num_segments=n)
        if i + 1 < num_layers:
            agg = jax.ops.segment_sum(m, dst, num_segments=n)
            h = hn + _silu(jnp.concatenate([hn, agg], axis=-1) @ Wh1[i]
                           + bh1[i]) @ Wh2[i] + bh2[i]
        x = x - x.mean(axis=0, keepdims=True)
    return x


# speedup vs baseline: 1.3562x; 1.3562x over previous
import functools

import jax
import jax.numpy as jnp
from jax import lax
from jax.experimental import pallas as pl
from jax.experimental.pallas import tpu as pltpu
from jax.experimental.pallas import tpu_sc as plsc

_N = 10000
_E = 320000
_H = 128
_NW = 32
_C = 80
_ROWS = _E // _C
_RPW = _ROWS // _NW


def _silu(x):
    return x * jax.nn.sigmoid(x)


def _make_gather2():
    mesh = plsc.VectorSubcoreMesh(core_axis_name="c", subcore_axis_name="s")

    @functools.partial(
        pl.kernel, mesh=mesh,
        out_type=[jax.ShapeDtypeStruct((_E, _H), jnp.float32),
                  jax.ShapeDtypeStruct((_E, _H), jnp.float32)],
        scratch_types=[
            pltpu.VMEM((_C,), jnp.int32),
            pltpu.VMEM((_C,), jnp.int32),
            pltpu.VMEM((_C, _H), jnp.float32),
            pltpu.VMEM((_C, _H), jnp.float32),
            pltpu.SemaphoreType.DMA,
            pltpu.SemaphoreType.DMA,
        ],
    )
    def gather2(ptab_hbm, qtab_hbm, src_hbm, dst_hbm, ps_out, qs_out,
                idxs_v, idxd_v, bufp_v, bufq_v, sem_p, sem_q):
        wid = lax.axis_index("s") * 2 + lax.axis_index("c")
        base = wid * _RPW

        @pl.loop(0, _RPW)
        def _(j):
            row = base + j
            pltpu.sync_copy(src_hbm.at[row], idxs_v)
            pltpu.sync_copy(dst_hbm.at[row], idxd_v)
            cp = pltpu.async_copy(ptab_hbm.at[idxs_v], bufp_v, sem_p)
            cq = pltpu.async_copy(qtab_hbm.at[idxd_v], bufq_v, sem_q)
            cp.wait()
            cq.wait()
            pltpu.sync_copy(bufp_v, ps_out.at[pl.ds(row * _C, _C)])
            pltpu.sync_copy(bufq_v, qs_out.at[pl.ds(row * _C, _C)])

    return gather2


_gather2 = _make_gather2()


_BE = 2000


def _edge_body(ps_ref, qs_ref, a_ref, we1a_ref, we2_ref,
               be2_ref, wx1_ref, bx1_ref, wx2_ref, bx2_ref, m_ref, w_ref):
    a = a_ref[...]
    m1 = _silu(ps_ref[...] + qs_ref[...] + a * we1a_ref[...])
    m = _silu(jnp.dot(m1, we2_ref[...],
                      preferred_element_type=jnp.float32) + be2_ref[...])
    t = _silu(jnp.dot(m, wx1_ref[...],
                      preferred_element_type=jnp.float32) + bx1_ref[...])
    w = jnp.dot(t, wx2_ref[...],
                preferred_element_type=jnp.float32) + bx2_ref[...]
    m_ref[...] = m
    w_ref[...] = w


def _edge_mlp(ps, qs, a, we1a, we2, be2, wx1, bx1, wx2, bx2):
    grid = (_E // _BE,)
    full = lambda i: (0, 0)
    in_specs = [
        pl.BlockSpec((_BE, _H), lambda i: (i, 0)),
        pl.BlockSpec((_BE, _H), lambda i: (i, 0)),
        pl.BlockSpec((_BE, 1), lambda i: (i, 0)),
        pl.BlockSpec((1, _H), full),
        pl.BlockSpec((_H, _H), full),
        pl.BlockSpec((1, _H), full),
        pl.BlockSpec((_H, _H), full),
        pl.BlockSpec((1, _H), full),
        pl.BlockSpec((_H, 1), full),
        pl.BlockSpec((1, 1), full),
    ]
    out_specs = [pl.BlockSpec((_BE, _H), lambda i: (i, 0)),
                 pl.BlockSpec((_BE, 1), lambda i: (i, 0))]
    out_shape = [jax.ShapeDtypeStruct((_E, _H), jnp.float32),
                 jax.ShapeDtypeStruct((_E, 1), jnp.float32)]
    return pl.pallas_call(
        _edge_body, grid=grid, in_specs=in_specs, out_specs=out_specs,
        out_shape=out_shape,
    )(ps, qs, a, we1a.reshape(1, _H), we2, be2.reshape(1, _H),
      wx1, bx1.reshape(1, _H), wx2, bx2.reshape(1, 1))


def kernel(atom_nums, coords, masses, masses_normalized, cond_labels,
           cond_mask, moments, temb, edge_index, embed, W_h, b_h, gamma,
           beta, We1, be1, We2, be2, Wx1, bx1, Wx2, bx2, Wh1, bh1, Wh2, bh2):
    n = coords.shape[0]
    src = edge_index[0]
    dst = edge_index[1]
    src2d = src.reshape(_ROWS, _C)
    dst2d = dst.reshape(_ROWS, _C)
    aemb = jnp.take(embed, atom_nums, axis=0)
    f = jnp.concatenate([aemb, temb, masses / 12.0, masses_normalized,
                         cond_labels, cond_mask, moments], axis=-1)
    h = f @ W_h + b_h
    a = jnp.sum((coords[src] - coords[dst]) ** 2, axis=-1, keepdims=True)
    x = coords
    num_layers = gamma.shape[0]
    c = 1.0 / float(edge_index.shape[1] // n)
    for i in range(num_layers):
        mu = h.mean(axis=-1, keepdims=True)
        var = h.var(axis=-1, keepdims=True)
        hn = (h - mu) / jnp.sqrt(var + 1e-5) * gamma[i] + beta[i]
        p = hn @ We1[i][:_H]
        q = hn @ We1[i][_H:2 * _H] + be1[i]
        ps, qs = _gather2(p, q, src2d, dst2d)
        m, w = _edge_mlp(ps, qs, a, We1[i][2 * _H], We2[i], be2[i],
                         Wx1[i], bx1[i], Wx2[i], bx2[i])
        delta = (x[dst] - x[src]) * w
        x = x + c * jax.ops.segment_sum(delta, dst, num_segments=n)
        if i + 1 < num_layers:
            agg = jax.ops.segment_sum(m, dst, num_segments=n)
            h = hn + _silu(jnp.concatenate([hn, agg], axis=-1) @ Wh1[i]
                           + bh1[i]) @ Wh2[i] + bh2[i]
        x = x - x.mean(axis=0, keepdims=True)
    return x


# speedup vs baseline: 2.9165x; 2.1505x over previous
import dataclasses
import functools

import jax
import jax.numpy as jnp
from jax import lax
from jax.experimental import pallas as pl
from jax.experimental.pallas import tpu as pltpu
from jax.experimental.pallas import tpu_sc as plsc

_N = 10000
_E = 320000
_H = 128
_WT = 256
_NW = 32
_C = 80
_ROWS = _E // _C
_RPW = _ROWS // _NW

@functools.lru_cache(maxsize=None)
def _get_mesh():
    return plsc.VectorSubcoreMesh(core_axis_name="c", subcore_axis_name="s")


def _get_cp():
    cp = pltpu.CompilerParams()
    if "needs_layout_passes" in pltpu.CompilerParams.__dataclass_fields__:
        cp = dataclasses.replace(cp, needs_layout_passes=False)
    return cp


def _silu(x):
    return x * jax.nn.sigmoid(x)


def _splat(v):
    return jnp.full((16,), v, jnp.int32)


@functools.lru_cache(maxsize=None)
def _make_gather2(wq):
    @functools.partial(
        pl.kernel, mesh=_get_mesh(),
        out_type=[jax.ShapeDtypeStruct((_E, _WT), jnp.float32),
                  jax.ShapeDtypeStruct((_E, wq), jnp.float32)],
        scratch_types=[
            pltpu.VMEM((_C,), jnp.int32),
            pltpu.VMEM((_C,), jnp.int32),
            pltpu.VMEM((_C, _WT), jnp.float32),
            pltpu.VMEM((_C, wq), jnp.float32),
            pltpu.SemaphoreType.DMA,
            pltpu.SemaphoreType.DMA,
        ],
    )
    def gather2(ptab_hbm, qtab_hbm, src_hbm, dst_hbm, ps_out, qs_out,
                idxs_v, idxd_v, bufp_v, bufq_v, sem_p, sem_q):
        wid = lax.axis_index("s") * 2 + lax.axis_index("c")
        base = wid * _RPW

        @pl.loop(0, _RPW)
        def _(j):
            row = base + j
            pltpu.sync_copy(src_hbm.at[row], idxs_v)
            pltpu.sync_copy(dst_hbm.at[row], idxd_v)
            cp = pltpu.async_copy(ptab_hbm.at[idxs_v], bufp_v, sem_p)
            cq = pltpu.async_copy(qtab_hbm.at[idxd_v], bufq_v, sem_q)
            cp.wait()
            cq.wait()
            pltpu.sync_copy(bufp_v, ps_out.at[pl.ds(row * _C, _C)])
            pltpu.sync_copy(bufq_v, qs_out.at[pl.ds(row * _C, _C)])

    return gather2


def _gather2_wide(*args):
    return _make_gather2(_WT)(*args)


def _gather2(*args):
    return _make_gather2(_H)(*args)


@functools.lru_cache(maxsize=None)
def _make_scatter(with_m):
    out_type = [jax.ShapeDtypeStruct((2, _N, 16), jnp.float32)]
    scratch = [
        pltpu.VMEM((_C,), jnp.int32),
        pltpu.VMEM((_C, 16), jnp.float32),
        pltpu.VMEM_SHARED((_N, 16), jnp.float32),
        pltpu.SemaphoreType.DMA,
    ]
    if with_m:
        out_type.append(jax.ShapeDtypeStruct((2, _N, _H), jnp.float32))
        scratch = scratch + [pltpu.VMEM((_C, _H), jnp.float32),
                             pltpu.VMEM_SHARED((_N, _H), jnp.float32)]

    @functools.partial(pl.kernel, mesh=_get_mesh(), out_type=out_type,
                       scratch_types=scratch, compiler_params=_get_cp())
    def scatter(*refs):
        if with_m:
            (vals_hbm, dst_hbm, z16_hbm, pay_hbm, zn_hbm,
             accw_out, accm_out,
             dstbuf_v, vals_v, accw_sh, sem,
             paybuf_v, accm_sh) = refs
        else:
            (vals_hbm, dst_hbm, z16_hbm,
             accw_out,
             dstbuf_v, vals_v, accw_sh, sem) = refs
        cid = lax.axis_index("c")
        sid = lax.axis_index("s")
        wid = sid * 2 + cid
        base = wid * _RPW

        @pl.when(sid == 0)
        def _():
            pltpu.sync_copy(z16_hbm, accw_sh)
            if with_m:
                pltpu.sync_copy(zn_hbm, accm_sh)
        plsc.subcore_barrier()

        @pl.loop(0, _RPW)
        def _(j):
            row = base + j
            if with_m:
                pltpu.sync_copy(pay_hbm.at[pl.ds(row * _C, _C)], paybuf_v)
            pltpu.sync_copy(dst_hbm.at[row], dstbuf_v)
            pltpu.sync_copy(vals_hbm.at[pl.ds(row * _C, _C)], vals_v)
            pltpu.sync_copy(vals_v, accw_sh.at[dstbuf_v], add=True)
            if with_m:
                pltpu.sync_copy(paybuf_v, accm_sh.at[dstbuf_v], add=True)

        plsc.subcore_barrier()

        @pl.when(sid == 0)
        def _():
            pltpu.sync_copy(accw_sh, accw_out.at[cid])
            if with_m:
                pltpu.sync_copy(accm_sh, accm_out.at[cid])

    return scatter


def _scatter_mw(*args):
    return _make_scatter(True)(*args)


def _scatter_w(*args):
    return _make_scatter(False)(*args)


_BE = 2000


def _edge_body(layer0, with_m, ps_ref, qs_ref, a_ref, we1a_ref, we2_ref,
               be2_ref, wx1_ref, bx1_ref, wx2_ref, bx2_ref, *out_refs):
    p = ps_ref[:, :_H]
    xs = ps_ref[:, _H:_H + 3]
    q = qs_ref[:, :_H]
    if layer0:
        xd = qs_ref[:, _H:_H + 3]
        a = jnp.sum((xs - xd) ** 2, axis=-1, keepdims=True)
    else:
        a = a_ref[...]
    m1 = _silu(p + q + a * we1a_ref[...])
    m = _silu(jnp.dot(m1, we2_ref[...],
                      preferred_element_type=jnp.float32) + be2_ref[...])
    t = _silu(jnp.dot(m, wx1_ref[...],
                      preferred_element_type=jnp.float32) + bx1_ref[...])
    w = jnp.dot(t, wx2_ref[...],
                preferred_element_type=jnp.float32) + bx2_ref[...]
    k = 0
    if with_m:
        out_refs[0][...] = m
        k = 1
    pad = jnp.zeros((w.shape[0], 12), jnp.float32)
    out_refs[k][...] = jnp.concatenate([w, w * xs, pad], axis=-1)
    if layer0:
        out_refs[k + 1][...] = a


def _edge_mlp(ps, qs, a, we1a, we2, be2, wx1, bx1, wx2, bx2,
              layer0, with_m):
    grid = (_E // _BE,)
    full = lambda i: (0, 0)
    wq = qs.shape[1]
    in_specs = [
        pl.BlockSpec((_BE, _WT), lambda i: (i, 0)),
        pl.BlockSpec((_BE, wq), lambda i: (i, 0)),
        pl.BlockSpec((_BE, 1), lambda i: (i, 0)),
        pl.BlockSpec((1, _H), full),
        pl.BlockSpec((_H, _H), full),
        pl.BlockSpec((1, _H), full),
        pl.BlockSpec((_H, _H), full),
        pl.BlockSpec((1, _H), full),
        pl.BlockSpec((_H, 1), full),
        pl.BlockSpec((1, 1), full),
    ]
    out_specs = []
    out_shape = []
    if with_m:
        out_specs.append(pl.BlockSpec((_BE, _H), lambda i: (i, 0)))
        out_shape.append(jax.ShapeDtypeStruct((_E, _H), jnp.float32))
    out_specs += [pl.BlockSpec((_BE, 16), lambda i: (i, 0))]
    out_shape += [jax.ShapeDtypeStruct((_E, 16), jnp.float32)]
    if layer0:
        out_specs.append(pl.BlockSpec((_BE, 1), lambda i: (i, 0)))
        out_shape.append(jax.ShapeDtypeStruct((_E, 1), jnp.float32))
    if a is None:
        a = jnp.zeros((_E, 1), jnp.float32)
    return pl.pallas_call(
        functools.partial(_edge_body, layer0, with_m),
        grid=grid, in_specs=in_specs, out_specs=out_specs,
        out_shape=out_shape,
    )(ps, qs, a, we1a.reshape(1, _H), we2, be2.reshape(1, _H),
      wx1, bx1.reshape(1, _H), wx2, bx2.reshape(1, 1))


def kernel(atom_nums, coords, masses, masses_normalized, cond_labels,
           cond_mask, moments, temb, edge_index, embed, W_h, b_h, gamma,
           beta, We1, be1, We2, be2, Wx1, bx1, Wx2, bx2, Wh1, bh1, Wh2, bh2):
    n = coords.shape[0]
    src2d = edge_index[0].reshape(_ROWS, _C)
    dst2d = edge_index[1].reshape(_ROWS, _C)
    aemb = jnp.take(embed, atom_nums, axis=0)
    f = jnp.concatenate([aemb, temb, masses / 12.0, masses_normalized,
                         cond_labels, cond_mask, moments], axis=-1)
    h = f @ W_h + b_h
    x = coords
    num_layers = gamma.shape[0]
    c = 1.0 / float(edge_index.shape[1] // n)
    z16 = jnp.zeros((n, 16), jnp.float32)
    zn = jnp.zeros((n, _H), jnp.float32)
    padt = jnp.zeros((n, _WT - _H - 3), jnp.float32)
    a = None
    for i in range(num_layers):
        mu = h.mean(axis=-1, keepdims=True)
        var = h.var(axis=-1, keepdims=True)
        hn = (h - mu) / jnp.sqrt(var + 1e-5) * gamma[i] + beta[i]
        p = hn @ We1[i][:_H]
        q = hn @ We1[i][_H:2 * _H] + be1[i]
        ptab = jnp.concatenate([p, x, padt], axis=-1)
        if i == 0:
            qtab = jnp.concatenate([q, x, padt], axis=-1)
            ps, qs = _gather2_wide(ptab, qtab, src2d, dst2d)
        else:
            ps, qs = _gather2(ptab, q, src2d, dst2d)
        last = i + 1 == num_layers
        out = list(_edge_mlp(ps, qs, a, We1[i][2 * _H], We2[i], be2[i],
                             Wx1[i], bx1[i], Wx2[i], bx2[i],
                             layer0=(i == 0), with_m=not last))
        m = out.pop(0) if not last else None
        vals16 = out.pop(0)
        if i == 0:
            a = out.pop(0)
        if last:
            accw = _scatter_w(vals16, dst2d, z16)[0]
        else:
            accw, accm = _scatter_mw(vals16, dst2d, z16, m, zn)
        if not last:
            agg = accm[0] + accm[1]
            h = hn + _silu(jnp.concatenate([hn, agg], axis=-1) @ Wh1[i]
                           + bh1[i]) @ Wh2[i] + bh2[i]
        aw = accw[0] + accw[1]
        sw = aw[:, :1]
        swx = aw[:, 1:4]
        x = x + c * (x * sw - swx)
        x = x - x.mean(axis=0, keepdims=True)
    return x
